# Initial kernel scaffold; baseline (speedup 1.0000x reference)
import jax
import jax.numpy as jnp
from jax.experimental import pallas as pl


def kernel(x, codebook):
    raise NotImplementedError("write your pallas kernel here")


# speedup vs baseline: 1.1479x; 1.1479x over previous
import functools

import jax
import jax.numpy as jnp
from jax import lax
from jax.experimental import pallas as pl
from jax.experimental.pallas import tpu as pltpu
from jax.experimental.pallas import tpu_sc as plsc

_D = 64
_K = 8192
_BETA = 0.25

_T = 256
_C = 1024
_KC = _K // _C

_NC = 2
_NS = 16
_NW = _NC * _NS
_GCHUNK = 128


def _dist_argmin_body(x_ref, cbt_ref, idx_ref, loss_ref):
    i = pl.program_id(0)
    xt = x_ref[0]
    x2 = jnp.sum(xt * xt, axis=1, keepdims=True)
    best = None
    bidx = None
    for kc in range(_KC):
        ebt = cbt_ref[:, kc * _C:(kc + 1) * _C]
        e2 = jnp.sum(ebt * ebt, axis=0, keepdims=True)
        xe = lax.dot_general(xt, ebt, (((1,), (0,)), ((), ())),
                             preferred_element_type=jnp.float32)
        dist = (x2 + e2) - 2.0 * xe
        vmin = jnp.min(dist, axis=1, keepdims=True)
        jidx = lax.broadcasted_iota(jnp.int32, (_T, _C), 1) + kc * _C
        masked = jnp.where(dist == vmin, jidx, jnp.int32(2147483647))
        imin = jnp.min(masked, axis=1, keepdims=True)
        if best is None:
            best, bidx = vmin, imin
        else:
            take = vmin < best
            bidx = jnp.where(take, imin, bidx)
            best = jnp.where(take, vmin, best)

    idx_ref[0] = bidx

    @pl.when(i == 0)
    def _():
        loss_ref[...] = jnp.zeros((1, 1), jnp.float32)

    loss_ref[...] += jnp.sum(best, keepdims=True)


def _dist_argmin(x3, cbt, *, interpret=False):
    nb = x3.shape[0]
    return pl.pallas_call(
        _dist_argmin_body,
        grid=(nb,),
        in_specs=[
            pl.BlockSpec((1, _T, _D), lambda i: (i, 0, 0)),
            pl.BlockSpec((_D, _K), lambda i: (0, 0)),
        ],
        out_specs=[
            pl.BlockSpec((1, _T, 1), lambda i: (i, 0, 0)),
            pl.BlockSpec((1, 1), lambda i: (0, 0)),
        ],
        out_shape=[
            jax.ShapeDtypeStruct((nb, _T, 1), jnp.int32),
            jax.ShapeDtypeStruct((1, 1), jnp.float32),
        ],
        interpret=interpret,
    )(x3, cbt)


def _gather_body(table_hbm, idx_hbm, out_hbm, idx_v, rows_v, sem):
    wid = lax.axis_index("s") * _NC + lax.axis_index("c")
    nchunk = idx_v.shape[0]
    bpw = nchunk * _GCHUNK
    pltpu.sync_copy(idx_hbm.at[wid], idx_v)
    copies = []
    for j in range(nchunk):
        copies.append(pltpu.async_copy(
            table_hbm.at[idx_v.at[j]],
            rows_v.at[pl.ds(j * _GCHUNK, _GCHUNK)],
            sem,
        ))
    for c in copies:
        c.wait()
    pltpu.sync_copy(rows_v, out_hbm.at[pl.ds(wid * bpw, bpw)])


def _make_gather(n_tokens):
    bpw = n_tokens // _NW
    nchunk = bpw // _GCHUNK
    mesh = plsc.VectorSubcoreMesh(core_axis_name="c", subcore_axis_name="s")
    return pl.kernel(
        _gather_body,
        out_type=jax.ShapeDtypeStruct((n_tokens, _D), jnp.float32),
        mesh=mesh,
        scratch_types=[
            pltpu.VMEM((nchunk, _GCHUNK), jnp.int32),
            pltpu.VMEM((bpw, _D), jnp.float32),
            pltpu.SemaphoreType.DMA,
        ],
        compiler_params=pltpu.CompilerParams(use_tc_tiling_on_sc=False),
    )


def kernel(x, codebook):
    xf = x.reshape(-1, _D)
    n = xf.shape[0]
    nb = n // _T
    x3 = xf.reshape(nb, _T, _D)
    cbt = codebook.T
    idx3, loss = _dist_argmin(x3, cbt)
    idx = idx3.reshape(n)
    idx_sc = idx.reshape(_NW, -1, _GCHUNK)
    x_q = _make_gather(n)(codebook, idx_sc).reshape(x.shape)
    vq_loss = (loss[0, 0] * jnp.float32((1.0 + _BETA) / (n * _D))).reshape(())
    return (x_q, idx, vq_loss)


# speedup vs baseline: 1.3095x; 1.1408x over previous
import jax
import jax.numpy as jnp
from jax import lax
from jax.experimental import pallas as pl
from jax.experimental.pallas import tpu as pltpu
from jax.experimental.pallas import tpu_sc as plsc

_D = 64
_K = 8192
_BETA = 0.25

_T = 256
_C = 1024
_KC = _K // _C

_NC = 2
_NS = 16
_NW = _NC * _NS
_GCHUNK = 128


def _dist_argmin_body(x_ref, cbt_ref, idx_ref, loss_ref):
    i = pl.program_id(0)
    xt = x_ref[0]
    x2 = jnp.sum(xt * xt, axis=1, keepdims=True)
    xm2 = xt * (-2.0)
    best = None
    bidx = None
    for kc in range(_KC):
        ebt = cbt_ref[:, kc * _C:(kc + 1) * _C]
        e2 = jnp.sum(ebt * ebt, axis=0, keepdims=True)
        xe = lax.dot_general(xm2, ebt, (((1,), (0,)), ((), ())),
                             preferred_element_type=jnp.float32)
        dist = xe + e2
        vmin = jnp.min(dist, axis=1, keepdims=True)
        jidx = lax.broadcasted_iota(jnp.int32, (_T, _C), 1) + kc * _C
        masked = jnp.where(dist == vmin, jidx, jnp.int32(2147483647))
        imin = jnp.min(masked, axis=1, keepdims=True)
        if best is None:
            best, bidx = vmin, imin
        else:
            take = vmin < best
            bidx = jnp.where(take, imin, bidx)
            best = jnp.where(take, vmin, best)

    idx_ref[0] = bidx

    @pl.when(i == 0)
    def _():
        loss_ref[...] = jnp.zeros((1, 1), jnp.float32)

    loss_ref[...] += jnp.sum(best + x2, keepdims=True)


def _dist_argmin(x3, cbt):
    nb = x3.shape[0]
    return pl.pallas_call(
        _dist_argmin_body,
        grid=(nb,),
        in_specs=[
            pl.BlockSpec((1, _T, _D), lambda i: (i, 0, 0)),
            pl.BlockSpec((_D, _K), lambda i: (0, 0)),
        ],
        out_specs=[
            pl.BlockSpec((1, _T, 1), lambda i: (i, 0, 0)),
            pl.BlockSpec((1, 1), lambda i: (0, 0)),
        ],
        out_shape=[
            jax.ShapeDtypeStruct((nb, _T, 1), jnp.int32),
            jax.ShapeDtypeStruct((1, 1), jnp.float32),
        ],
    )(x3, cbt)


def _gather_body(table_hbm, idx_hbm, out_hbm, idx_v, rows_v, sem):
    wid = lax.axis_index("s") * _NC + lax.axis_index("c")
    nchunk = idx_v.shape[0]
    bpw = nchunk * _GCHUNK
    pltpu.sync_copy(idx_hbm.at[wid], idx_v)
    copies = []
    for j in range(nchunk):
        copies.append(pltpu.async_copy(
            table_hbm.at[idx_v.at[j]],
            rows_v.at[pl.ds(j * _GCHUNK, _GCHUNK)],
            sem,
        ))
    for c in copies:
        c.wait()
    pltpu.sync_copy(rows_v, out_hbm.at[pl.ds(wid * bpw, bpw)])


def _make_gather(n_tokens):
    bpw = n_tokens // _NW
    nchunk = bpw // _GCHUNK
    mesh = plsc.VectorSubcoreMesh(core_axis_name="c", subcore_axis_name="s")
    return pl.kernel(
        _gather_body,
        out_type=jax.ShapeDtypeStruct((n_tokens, _D), jnp.float32),
        mesh=mesh,
        scratch_types=[
            pltpu.VMEM((nchunk, _GCHUNK), jnp.int32),
            pltpu.VMEM((bpw, _D), jnp.float32),
            pltpu.SemaphoreType.DMA,
        ],
        compiler_params=pltpu.CompilerParams(use_tc_tiling_on_sc=False),
    )


def kernel(x, codebook):
    xf = x.reshape(-1, _D)
    n = xf.shape[0]
    nb = n // _T
    x3 = xf.reshape(nb, _T, _D)
    cbt = codebook.T
    idx3, loss = _dist_argmin(x3, cbt)
    idx = idx3.reshape(n)
    idx_sc = idx.reshape(_NW, -1, _GCHUNK)
    x_q = _make_gather(n)(codebook, idx_sc).reshape(x.shape)
    vq_loss = (loss[0, 0] * jnp.float32((1.0 + _BETA) / (n * _D))).reshape(())
    return (x_q, idx, vq_loss)
